# Initial kernel scaffold; baseline (speedup 1.0000x reference)
import jax
import jax.numpy as jnp
from jax.experimental import pallas as pl


def kernel(x, edge_index, W1l, b1, W1r, W2l, b2, W2r):
    raise NotImplementedError("write your pallas kernel here")


# speedup vs baseline: 9.2510x; 9.2510x over previous
import jax
import jax.numpy as jnp
from jax import lax
from jax.experimental import pallas as pl
from jax.experimental.pallas import tpu as pltpu
from jax.experimental.pallas import tpu_sc as plsc

N = 10000
E = 320000
D = 128
DH = D // 2
NC = 2
NS = 16
EPW = E // NS
K = 100
CH = EPW // K
CHH = CH // 2
NP = 10240
RPS = NP // NS
CW = 16


def _sc_pass(with_counts: bool):
  out_type = [jax.ShapeDtypeStruct((NC, NP, DH), jnp.float32)]
  scratch = [
      pltpu.VMEM((CH, K), jnp.int32),
      pltpu.VMEM((CH, K), jnp.int32),
      pltpu.VMEM((K, DH), jnp.float32),
      pltpu.VMEM((K, DH), jnp.float32),
      pltpu.VMEM_SHARED((NP, DH), jnp.float32),
      pltpu.SemaphoreType.DMA,
      pltpu.SemaphoreType.DMA,
  ]
  if with_counts:
    out_type.append(jax.ShapeDtypeStruct((NC, NP, CW), jnp.float32))
    scratch += [
        pltpu.VMEM((K, CW), jnp.float32),
        pltpu.VMEM_SHARED((NP, CW), jnp.float32),
    ]

  mesh = plsc.VectorSubcoreMesh(core_axis_name="c", subcore_axis_name="s")

  def body(table, srcs, dsts, ones_h, zeros_h, zeros_c, *rest):
    if with_counts:
      (agg_out, cnt_out, src_v, dst_v, rows0, rows1, agg_sh, sem0, sem1,
       ones_v, cnt_sh) = rest
    else:
      agg_out, src_v, dst_v, rows0, rows1, agg_sh, sem0, sem1 = rest
    cid = lax.axis_index("c")
    sid = lax.axis_index("s")

    pltpu.sync_copy(srcs.at[cid * NS + sid], src_v)
    pltpu.sync_copy(dsts.at[sid], dst_v)
    row_sl = pl.ds(sid * RPS, RPS)
    pltpu.sync_copy(zeros_h.at[row_sl], agg_sh.at[row_sl])
    if with_counts:
      pltpu.sync_copy(ones_h, ones_v)
      pltpu.sync_copy(zeros_c.at[row_sl], cnt_sh.at[row_sl])
    plsc.subcore_barrier()

    rows = (rows0, rows1)
    sems = (sem0, sem1)
    pltpu.async_copy(table.at[src_v.at[0]], rows0, sem0)

    def chunk(j, _):
      for b in range(2):
        jj = j + b

        @pl.when(jj + 1 < CH)
        def _():
          pltpu.async_copy(table.at[src_v.at[jj + 1]], rows[1 - b],
                           sems[1 - b])
        pltpu.make_async_copy(table.at[src_v.at[jj]], rows[b], sems[b]).wait()
        pltpu.sync_copy(rows[b], agg_sh.at[dst_v.at[jj]], add=True)
        if with_counts:
          @pl.when(lax.select(cid == 0, jj < CHH, jj >= CHH))
          def _():
            pltpu.sync_copy(ones_v, cnt_sh.at[dst_v.at[jj]], add=True)
      return 0

    lax.fori_loop(0, CH // 2, lambda i, c: chunk(i * 2, c), 0)
    plsc.subcore_barrier()

    pltpu.sync_copy(agg_sh.at[row_sl], agg_out.at[cid].at[row_sl])
    if with_counts:
      pltpu.sync_copy(cnt_sh.at[row_sl], cnt_out.at[cid].at[row_sl])

  return pl.kernel(body, out_type=tuple(out_type), mesh=mesh,
                   scratch_types=scratch,
                   compiler_params=pltpu.CompilerParams(
                       use_tc_tiling_on_sc=False))


_sc_agg_counts = _sc_pass(with_counts=True)
_sc_agg = _sc_pass(with_counts=False)


def _tc_combine(relu: bool):
  BLK = 2000
  GRID = N // BLK

  def body(aggp, cntp, x, wlt, wrt, b, o):
    cnt = cntp[0, :, 0:1] + cntp[1, :, 0:1]
    recip = 1.0 / jnp.maximum(cnt, 1.0)
    mean_l = aggp[0] * recip
    mean_r = aggp[1] * recip
    y = (jnp.dot(mean_l, wlt[0:DH, :], preferred_element_type=jnp.float32)
         + jnp.dot(mean_r, wlt[DH:D, :], preferred_element_type=jnp.float32)
         + jnp.dot(x[...], wrt[...], preferred_element_type=jnp.float32)
         + b[...])
    o[...] = jnp.maximum(y, 0.0) if relu else y

  return pl.pallas_call(
      body,
      grid=(GRID,),
      in_specs=[
          pl.BlockSpec((NC, BLK, DH), lambda i: (0, i, 0)),
          pl.BlockSpec((NC, BLK, CW), lambda i: (0, i, 0)),
          pl.BlockSpec((BLK, D), lambda i: (i, 0)),
          pl.BlockSpec((D, D), lambda i: (0, 0)),
          pl.BlockSpec((D, D), lambda i: (0, 0)),
          pl.BlockSpec((1, D), lambda i: (0, 0)),
      ],
      out_specs=pl.BlockSpec((BLK, D), lambda i: (i, 0)),
      out_shape=jax.ShapeDtypeStruct((N, D), jnp.float32),
  )


_tc_combine_relu = _tc_combine(relu=True)
_tc_combine_lin = _tc_combine(relu=False)


def _split_table(t):
  return jnp.concatenate([t[:, :DH], t[:, DH:]], axis=0)


def kernel(x, edge_index, W1l, b1, W1r, W2l, b2, W2r):
  src = edge_index[0].reshape(NS, CH, K)
  dst = edge_index[1].reshape(NS, CH, K)
  src2 = jnp.concatenate([src, src + N], axis=0)
  ones_h = jnp.ones((K, CW), jnp.float32)
  zeros_h = jnp.zeros((NP, DH), jnp.float32)
  zeros_c = jnp.zeros((NP, CW), jnp.float32)

  agg1, cnt = _sc_agg_counts(_split_table(x), src2, dst, ones_h, zeros_h,
                             zeros_c)
  h = _tc_combine_relu(agg1, cnt, x, W1l.T, W1r.T, b1.reshape(1, D))
  (agg2,) = _sc_agg(_split_table(h), src2, dst, ones_h, zeros_h, zeros_c)
  out = _tc_combine_lin(agg2, cnt, h, W2l.T, W2r.T, b2.reshape(1, D))
  return out


# speedup vs baseline: 11.5850x; 1.2523x over previous
import jax
import jax.numpy as jnp
from jax import lax
from jax.experimental import pallas as pl
from jax.experimental.pallas import tpu as pltpu
from jax.experimental.pallas import tpu_sc as plsc

N = 10000
E = 320000
D = 128
DH = D // 2
NC = 2
NS = 16
EPW = E // NS
K = 100
CH = EPW // K
CHH = CH // 2
NP = 10240
RPS = NP // NS
CW = 16
NB = 4


def _sc_pass(with_counts: bool):
  out_type = [jax.ShapeDtypeStruct((NC, NP, DH), jnp.float32)]
  scratch = [
      pltpu.VMEM((CH, K), jnp.int32),
      pltpu.VMEM((CH, K), jnp.int32),
  ]
  scratch += [pltpu.VMEM((K, DH), jnp.float32) for _ in range(NB)]
  scratch += [pltpu.SemaphoreType.DMA for _ in range(2 * NB)]
  if with_counts:
    out_type.append(jax.ShapeDtypeStruct((NC, NP, CW), jnp.float32))
    scratch += [pltpu.VMEM((K, CW), jnp.float32)]
  scratch += [pltpu.VMEM_SHARED((NP, DH), jnp.float32)]
  if with_counts:
    scratch += [pltpu.VMEM_SHARED((NP, CW), jnp.float32)]

  mesh = plsc.VectorSubcoreMesh(core_axis_name="c", subcore_axis_name="s")

  def body(table, srcs, dsts, ones_h, zeros_h, zeros_c, *rest):
    if with_counts:
      agg_out, cnt_out = rest[0], rest[1]
      rest = rest[2:]
    else:
      agg_out = rest[0]
      rest = rest[1:]
    src_v, dst_v = rest[0], rest[1]
    bufs = rest[2:2 + NB]
    sem_g = rest[2 + NB:2 + 2 * NB]
    sem_s = rest[2 + 2 * NB:2 + 3 * NB]
    rest = rest[2 + 3 * NB:]
    if with_counts:
      ones_v, agg_sh, cnt_sh = rest
    else:
      (agg_sh,) = rest
    cid = lax.axis_index("c")
    sid = lax.axis_index("s")

    pltpu.sync_copy(srcs.at[cid * NS + sid], src_v)
    pltpu.sync_copy(dsts.at[sid], dst_v)
    row_sl = pl.ds(sid * RPS, RPS)
    pltpu.sync_copy(zeros_h.at[row_sl], agg_sh.at[row_sl])
    if with_counts:
      pltpu.sync_copy(ones_h, ones_v)
      pltpu.sync_copy(zeros_c.at[row_sl], cnt_sh.at[row_sl])
    plsc.subcore_barrier()

    def gather(jj, b):
      return pltpu.async_copy(table.at[src_v.at[jj]], bufs[b], sem_g[b])

    def scatter(jj, b):
      return pltpu.async_copy(bufs[b], agg_sh.at[dst_v.at[jj]], sem_s[b],
                              add=True)

    for b in range(NB):
      gather(b, b)

    def step(j, _):
      for b in range(NB):
        jj = j * NB + b
        pltpu.make_async_copy(table.at[src_v.at[jj]], bufs[b],
                              sem_g[b]).wait()
        scatter(jj, b)
        if with_counts:
          @pl.when(lax.select(cid == 0, jj < CHH, jj >= CHH))
          def _():
            pltpu.sync_copy(ones_v, cnt_sh.at[dst_v.at[jj]], add=True)
        bp = (b + NB - 1) % NB

        @pl.when(jnp.logical_and(jj >= 1, jj + NB - 1 < CH))
        def _():
          pltpu.make_async_copy(bufs[bp], agg_sh.at[dst_v.at[jj - 1]],
                                sem_s[bp]).wait()
          gather(jj + NB - 1, bp)
      return 0

    lax.fori_loop(0, CH // NB, step, 0)
    for b in range(NB):
      m = CH - NB + b
      pltpu.make_async_copy(bufs[m % NB], agg_sh.at[dst_v.at[m]],
                            sem_s[m % NB]).wait()
    plsc.subcore_barrier()

    pltpu.sync_copy(agg_sh.at[row_sl], agg_out.at[cid].at[row_sl])
    if with_counts:
      pltpu.sync_copy(cnt_sh.at[row_sl], cnt_out.at[cid].at[row_sl])

  return pl.kernel(body, out_type=tuple(out_type), mesh=mesh,
                   scratch_types=scratch,
                   compiler_params=pltpu.CompilerParams(
                       use_tc_tiling_on_sc=False))


_sc_agg_counts = _sc_pass(with_counts=True)
_sc_agg = _sc_pass(with_counts=False)


def _tc_combine(relu: bool, split_in: bool, split_out: bool):
  BLK = 2000
  GRID = N // BLK

  def body(aggp, cntp, x, wlt, wrt, b, o):
    cnt = cntp[0, :, 0:1] + cntp[1, :, 0:1]
    recip = 1.0 / jnp.maximum(cnt, 1.0)
    mean_l = aggp[0] * recip
    mean_r = aggp[1] * recip
    y = (jnp.dot(mean_l, wlt[0:DH, :], preferred_element_type=jnp.float32)
         + jnp.dot(mean_r, wlt[DH:D, :], preferred_element_type=jnp.float32)
         + b[...])
    if split_in:
      y += (jnp.dot(x[0], wrt[0:DH, :], preferred_element_type=jnp.float32)
            + jnp.dot(x[1], wrt[DH:D, :], preferred_element_type=jnp.float32))
    else:
      y += jnp.dot(x[...], wrt[...], preferred_element_type=jnp.float32)
    if relu:
      y = jnp.maximum(y, 0.0)
    if split_out:
      o[0] = y[:, 0:DH]
      o[1] = y[:, DH:D]
    else:
      o[...] = y

  x_spec = (pl.BlockSpec((NC, BLK, DH), lambda i: (0, i, 0)) if split_in
            else pl.BlockSpec((BLK, D), lambda i: (i, 0)))
  if split_out:
    out_spec = pl.BlockSpec((NC, BLK, DH), lambda i: (0, i, 0))
    out_shape = jax.ShapeDtypeStruct((NC, N, DH), jnp.float32)
  else:
    out_spec = pl.BlockSpec((BLK, D), lambda i: (i, 0))
    out_shape = jax.ShapeDtypeStruct((N, D), jnp.float32)

  return pl.pallas_call(
      body,
      grid=(GRID,),
      in_specs=[
          pl.BlockSpec((NC, BLK, DH), lambda i: (0, i, 0)),
          pl.BlockSpec((NC, BLK, CW), lambda i: (0, i, 0)),
          x_spec,
          pl.BlockSpec((D, D), lambda i: (0, 0)),
          pl.BlockSpec((D, D), lambda i: (0, 0)),
          pl.BlockSpec((1, D), lambda i: (0, 0)),
      ],
      out_specs=out_spec,
      out_shape=out_shape,
  )


_tc_combine1 = _tc_combine(relu=True, split_in=False, split_out=True)
_tc_combine2 = _tc_combine(relu=False, split_in=True, split_out=False)


def kernel(x, edge_index, W1l, b1, W1r, W2l, b2, W2r):
  src = edge_index[0].reshape(NS, CH, K)
  dst = edge_index[1].reshape(NS, CH, K)
  src2 = jnp.concatenate([src, src + N], axis=0)
  ones_h = jnp.ones((K, CW), jnp.float32)
  zeros_h = jnp.zeros((NP, DH), jnp.float32)
  zeros_c = jnp.zeros((NP, CW), jnp.float32)
  xs = jnp.concatenate([x[:, :DH], x[:, DH:]], axis=0)

  agg1, cnt = _sc_agg_counts(xs, src2, dst, ones_h, zeros_h, zeros_c)
  h2 = _tc_combine1(agg1, cnt, x, W1l.T, W1r.T, b1.reshape(1, D))
  (agg2,) = _sc_agg(h2.reshape(NC * N, DH), src2, dst, ones_h, zeros_h,
                    zeros_c)
  out = _tc_combine2(agg2, cnt, h2, W2l.T, W2r.T, b2.reshape(1, D))
  return out


# speedup vs baseline: 13.1566x; 1.1357x over previous
import jax
import jax.numpy as jnp
from jax import lax
from jax.experimental import pallas as pl
from jax.experimental.pallas import tpu as pltpu
from jax.experimental.pallas import tpu_sc as plsc

N = 10000
E = 320000
D = 128
DH = D // 2
NC = 2
NS = 16
EPW = E // NS
K = 100
CH = EPW // K
CHH = CH // 2
NP = 10240
RPS = NP // NS
CW = 16
NB = 4
ZR = 128


def _sc_pass(with_counts: bool):
  out_type = [jax.ShapeDtypeStruct((NC, NP, DH), jnp.float32)]
  scratch = [
      pltpu.VMEM((CH, K), jnp.int32),
      pltpu.VMEM((CH, K), jnp.int32),
      pltpu.VMEM((ZR, DH), jnp.float32),
  ]
  scratch += [pltpu.VMEM((K, DH), jnp.float32) for _ in range(NB)]
  scratch += [pltpu.SemaphoreType.DMA for _ in range(2 * NB)]
  if with_counts:
    out_type.append(jax.ShapeDtypeStruct((NC, NP, CW), jnp.float32))
    scratch += [pltpu.VMEM((K, CW), jnp.float32)]
  scratch += [pltpu.VMEM_SHARED((NP, DH), jnp.float32)]
  if with_counts:
    scratch += [pltpu.VMEM_SHARED((NP, CW), jnp.float32)]

  mesh = plsc.VectorSubcoreMesh(core_axis_name="c", subcore_axis_name="s")

  def body(table, srcs, dsts, ones_h, *rest):
    if with_counts:
      agg_out, cnt_out = rest[0], rest[1]
      rest = rest[2:]
    else:
      agg_out = rest[0]
      rest = rest[1:]
    src_v, dst_v, zbuf = rest[0], rest[1], rest[2]
    bufs = rest[3:3 + NB]
    sem_g = rest[3 + NB:3 + 2 * NB]
    sem_s = rest[3 + 2 * NB:3 + 3 * NB]
    rest = rest[3 + 3 * NB:]
    if with_counts:
      ones_v, agg_sh, cnt_sh = rest
    else:
      (agg_sh,) = rest
    cid = lax.axis_index("c")
    sid = lax.axis_index("s")

    pltpu.sync_copy(srcs.at[cid * NS + sid], src_v)
    pltpu.sync_copy(dsts.at[sid], dst_v)
    if with_counts:
      pltpu.sync_copy(ones_h, ones_v)
    z16 = jnp.zeros((16,), jnp.float32)

    def zrow(r, _):
      for c in range(DH // 16):
        zbuf[r, pl.ds(c * 16, 16)] = z16
      return 0

    lax.fori_loop(0, ZR, zrow, 0)
    for r in range(RPS // ZR):
      pltpu.sync_copy(zbuf, agg_sh.at[pl.ds(sid * RPS + r * ZR, ZR)])
    if with_counts:
      for r in range(RPS // ZR):
        pltpu.sync_copy(zbuf.at[:, pl.ds(0, CW)],
                        cnt_sh.at[pl.ds(sid * RPS + r * ZR, ZR)])
    plsc.subcore_barrier()

    def gather(jj, b):
      return pltpu.async_copy(table.at[src_v.at[jj]], bufs[b], sem_g[b])

    def scatter(jj, b):
      return pltpu.async_copy(bufs[b], agg_sh.at[dst_v.at[jj]], sem_s[b],
                              add=True)

    for b in range(NB):
      gather(b, b)

    def step(j, _):
      for b in range(NB):
        jj = j * NB + b
        pltpu.make_async_copy(table.at[src_v.at[jj]], bufs[b],
                              sem_g[b]).wait()
        scatter(jj, b)
        if with_counts:
          @pl.when(lax.select(cid == 0, jj < CHH, jj >= CHH))
          def _():
            pltpu.sync_copy(ones_v, cnt_sh.at[dst_v.at[jj]], add=True)
        bp = (b + NB - 1) % NB

        @pl.when(jnp.logical_and(jj >= 1, jj + NB - 1 < CH))
        def _():
          pltpu.make_async_copy(bufs[bp], agg_sh.at[dst_v.at[jj - 1]],
                                sem_s[bp]).wait()
          gather(jj + NB - 1, bp)
      return 0

    lax.fori_loop(0, CH // NB, step, 0)
    for b in range(NB):
      m = CH - NB + b
      pltpu.make_async_copy(bufs[m % NB], agg_sh.at[dst_v.at[m]],
                            sem_s[m % NB]).wait()
    plsc.subcore_barrier()

    row_sl = pl.ds(sid * RPS, RPS)
    pltpu.sync_copy(agg_sh.at[row_sl], agg_out.at[cid].at[row_sl])
    if with_counts:
      pltpu.sync_copy(cnt_sh.at[row_sl], cnt_out.at[cid].at[row_sl])

  return pl.kernel(body, out_type=tuple(out_type), mesh=mesh,
                   scratch_types=scratch,
                   compiler_params=pltpu.CompilerParams(
                       use_tc_tiling_on_sc=False))


_sc_agg_counts = _sc_pass(with_counts=True)
_sc_agg = _sc_pass(with_counts=False)


def _tc_combine(relu: bool):
  BLK = 2000
  GRID = N // BLK

  def body(aggp, cntp, x, wlt, wrt, b, o):
    cnt = cntp[0, :, 0:1] + cntp[1, :, 0:1]
    recip = 1.0 / jnp.maximum(cnt, 1.0)
    mean_l = aggp[0] * recip
    mean_r = aggp[1] * recip
    y = (jnp.dot(mean_l, wlt[0:DH, :], preferred_element_type=jnp.float32)
         + jnp.dot(mean_r, wlt[DH:D, :], preferred_element_type=jnp.float32)
         + jnp.dot(x[...], wrt[...], preferred_element_type=jnp.float32)
         + b[...])
    o[...] = jnp.maximum(y, 0.0) if relu else y

  return pl.pallas_call(
      body,
      grid=(GRID,),
      in_specs=[
          pl.BlockSpec((NC, BLK, DH), lambda i: (0, i, 0)),
          pl.BlockSpec((NC, BLK, CW), lambda i: (0, i, 0)),
          pl.BlockSpec((BLK, D), lambda i: (i, 0)),
          pl.BlockSpec((D, D), lambda i: (0, 0)),
          pl.BlockSpec((D, D), lambda i: (0, 0)),
          pl.BlockSpec((1, D), lambda i: (0, 0)),
      ],
      out_specs=pl.BlockSpec((BLK, D), lambda i: (i, 0)),
      out_shape=jax.ShapeDtypeStruct((N, D), jnp.float32),
  )


_tc_combine1 = _tc_combine(relu=True)
_tc_combine2 = _tc_combine(relu=False)


def kernel(x, edge_index, W1l, b1, W1r, W2l, b2, W2r):
  src = edge_index[0]
  dst = edge_index[1]
  src2 = jnp.concatenate([src * 2, src * 2 + 1]).reshape(NC * NS, CH, K)
  dst3 = dst.reshape(NS, CH, K)
  ones_h = jnp.ones((K, CW), jnp.float32)

  agg1, cnt = _sc_agg_counts(x.reshape(NC * N, DH), src2, dst3, ones_h)
  h = _tc_combine1(agg1, cnt, x, W1l.T, W1r.T, b1.reshape(1, D))
  (agg2,) = _sc_agg(h.reshape(NC * N, DH), src2, dst3, ones_h)
  out = _tc_combine2(agg2, cnt, h, W2l.T, W2r.T, b2.reshape(1, D))
  return out
